# Initial kernel scaffold; baseline (speedup 1.0000x reference)
import jax
import jax.numpy as jnp
from jax.experimental import pallas as pl


def kernel(feat, edge_index, edge_weight, W, b_fc, bias, coef_self, coef_posi, coef_nega):
    raise NotImplementedError("write your pallas kernel here")


# speedup vs baseline: 25.2825x; 25.2825x over previous
import functools

import jax
import jax.numpy as jnp
from jax import lax
from jax.experimental import pallas as pl
from jax.experimental.pallas import tpu as pltpu
from jax.experimental.pallas import tpu_sc as plsc

NC = 2
NS = 16
CHUNK = 80


def _mm_body(f_ref, wt_ref, bsum_ref, base_ref, g_ref):
    f = f_ref[...]
    wt = wt_ref[...]
    din = f.shape[1]
    base_ref[...] = (
        jnp.dot(f, wt[0:din], preferred_element_type=jnp.float32) + bsum_ref[...]
    )
    g_ref[0] = jnp.dot(f, wt[din : 2 * din], preferred_element_type=jnp.float32)
    g_ref[1] = jnp.dot(f, wt[2 * din : 3 * din], preferred_element_type=jnp.float32)


def _comb_body(b_ref, p0_ref, p1_ref, o_ref):
    o_ref[...] = b_ref[...] + p0_ref[...] + p1_ref[...]


def _edge_vals(wv, dv, n):
    zf = jnp.zeros((16,), jnp.float32)
    zi = jnp.zeros((16,), jnp.int32)
    nvec = jnp.full((16,), n, jnp.int32)
    negv = wv < zf
    ni = jnp.where(negv, nvec, zi)
    ev = jnp.where(wv > zf, jnp.exp(wv), jnp.where(negv, jnp.exp(-wv), zf))
    return ev, dv + ni, ni


def _make_sc_kernel(n, e, d):
    ea = e // NS
    eb = e // (NC * NS)
    rpt = (-(-n // NS) + 39) // 40 * 40
    npad = NS * rpt
    sp = ((2 * n + NS * 16 - 1) // (NS * 16)) * NS * 16
    spt = sp // NS
    mesh = plsc.VectorSubcoreMesh(core_axis_name="c", subcore_axis_name="s")

    stage = 2000

    def body(g_hbm, src_hbm, dst_hbm, w_hbm, znd_hbm, out_hbm,
             ws, ds_, ss, ebuf, d2buf, zbuf,
             dstbuf, d2bufb, sidxbuf, ebufb, denbuf,
             rows, s_tab, acc, sem):
        c = lax.axis_index("c")
        s = lax.axis_index("s")
        wid = c * NS + s

        zv = jnp.zeros((16,), jnp.float32)

        def zero_body(i, _):
            zbuf[pl.ds(i * 16, 16)] = zv
            return 0

        lax.fori_loop(0, spt // 16, zero_body, 0)
        pltpu.sync_copy(zbuf, s_tab.at[pl.ds(s * spt, spt)])
        pltpu.sync_copy(znd_hbm.at[pl.ds(s * rpt, rpt)],
                        acc.at[pl.ds(s * rpt, rpt)])

        plsc.subcore_barrier()

        def chunk_a(i, _):
            off = i * CHUNK
            for j in range(CHUNK // 16):
                wv = ws[pl.ds(off + j * 16, 16)]
                dv = ds_[pl.ds(off + j * 16, 16)]
                ev, d2v, _ = _edge_vals(wv, dv, n)
                ebuf[pl.ds(j * 16, 16)] = ev
                d2buf[pl.ds(j * 16, 16)] = d2v
            pltpu.sync_copy(ebuf, s_tab.at[d2buf], add=True)
            return 0

        def stage_a(t, _):
            sbase = s * ea + t * stage
            pltpu.sync_copy(w_hbm.at[pl.ds(sbase, stage)], ws)
            pltpu.sync_copy(dst_hbm.at[pl.ds(sbase, stage)], ds_)
            lax.fori_loop(0, stage // CHUNK, chunk_a, 0)
            return 0

        lax.fori_loop(0, ea // stage, stage_a, 0)

        plsc.subcore_barrier()

        def chunk_b(k, _):
            off = k * CHUNK
            for j in range(CHUNK // 16):
                wv = ws[pl.ds(off + j * 16, 16)]
                dv = ds_[pl.ds(off + j * 16, 16)]
                srcv = ss[pl.ds(off + j * 16, 16)]
                ev, d2v, ni = _edge_vals(wv, dv, n)
                ebufb[pl.ds(j * 16, 16)] = ev
                d2bufb[pl.ds(j * 16, 16)] = d2v
                sidxbuf[pl.ds(j * 16, 16)] = srcv + ni
                dstbuf[pl.ds(j * 16, 16)] = dv
            pltpu.sync_copy(s_tab.at[d2bufb], denbuf)
            pltpu.async_copy(g_hbm.at[sidxbuf], rows, sem).wait()
            tiny = jnp.full((16,), 1e-30, jnp.float32)
            for j in range(CHUNK // 16):
                ev = ebufb[pl.ds(j * 16, 16)]
                denv = denbuf[pl.ds(j * 16, 16)]
                av = ev / jnp.maximum(denv, tiny)
                for l in range(16):
                    sv = lax.gather(
                        av, jnp.full((16, 1), l, jnp.int32),
                        dimension_numbers=lax.GatherDimensionNumbers(
                            offset_dims=(), collapsed_slice_dims=(0,),
                            start_index_map=(0,)),
                        slice_sizes=(1,),
                        mode=lax.GatherScatterMode.PROMISE_IN_BOUNDS)
                    r = j * 16 + l
                    for t in range(d // 16):
                        rows[r, pl.ds(t * 16, 16)] = (
                            rows[r, pl.ds(t * 16, 16)] * sv)
            pltpu.sync_copy(rows, acc.at[dstbuf], add=True)
            return 0

        def stage_b(t, _):
            sbase = wid * eb + t * stage
            pltpu.sync_copy(w_hbm.at[pl.ds(sbase, stage)], ws)
            pltpu.sync_copy(dst_hbm.at[pl.ds(sbase, stage)], ds_)
            pltpu.sync_copy(src_hbm.at[pl.ds(sbase, stage)], ss)
            lax.fori_loop(0, stage // CHUNK, chunk_b, 0)
            return 0

        lax.fori_loop(0, eb // stage, stage_b, 0)

        plsc.subcore_barrier()
        pltpu.sync_copy(acc.at[pl.ds(s * rpt, rpt)],
                        out_hbm.at[pl.ds(c * npad + s * rpt, rpt)])

    return pl.kernel(
        body,
        out_type=jax.ShapeDtypeStruct((NC * npad, d), jnp.float32),
        mesh=mesh,
        scratch_types=[
            pltpu.VMEM((stage,), jnp.float32),
            pltpu.VMEM((stage,), jnp.int32),
            pltpu.VMEM((stage,), jnp.int32),
            pltpu.VMEM((CHUNK,), jnp.float32),
            pltpu.VMEM((CHUNK,), jnp.int32),
            pltpu.VMEM((spt,), jnp.float32),
            pltpu.VMEM((CHUNK,), jnp.int32),
            pltpu.VMEM((CHUNK,), jnp.int32),
            pltpu.VMEM((CHUNK,), jnp.int32),
            pltpu.VMEM((CHUNK,), jnp.float32),
            pltpu.VMEM((CHUNK,), jnp.float32),
            pltpu.VMEM((CHUNK, d), jnp.float32),
            pltpu.VMEM_SHARED((sp,), jnp.float32),
            pltpu.VMEM_SHARED((npad, d), jnp.float32),
            pltpu.SemaphoreType.DMA,
        ],
    )


def kernel(feat, edge_index, edge_weight, W, b_fc, bias,
           coef_self, coef_posi, coef_nega):
    n, din = feat.shape
    dout = W.shape[0]
    e = edge_weight.shape[0]

    wt = W.T
    wts = jnp.concatenate(
        [wt[:din] * coef_self, wt[din:2 * din] * coef_posi,
         wt[2 * din:] * coef_nega], axis=0)
    bsum = (b_fc + bias).reshape(1, dout)

    rb = 1000
    grid = (n // rb,)
    base, g = pl.pallas_call(
        _mm_body,
        grid=grid,
        in_specs=[
            pl.BlockSpec((rb, din), lambda i: (i, 0)),
            pl.BlockSpec((3 * din, dout), lambda i: (0, 0)),
            pl.BlockSpec((1, dout), lambda i: (0, 0)),
        ],
        out_specs=[
            pl.BlockSpec((rb, dout), lambda i: (i, 0)),
            pl.BlockSpec((2, rb, dout), lambda i: (0, i, 0)),
        ],
        out_shape=[
            jax.ShapeDtypeStruct((n, dout), jnp.float32),
            jax.ShapeDtypeStruct((2, n, dout), jnp.float32),
        ],
    )(feat, wts, bsum)

    src = edge_index[0]
    dst = edge_index[1]
    rpt = (-(-n // NS) + 39) // 40 * 40
    npad = NS * rpt
    znd = jnp.zeros((npad, dout), jnp.float32)
    sc = _make_sc_kernel(n, e, dout)
    partial = sc(g.reshape(2 * n, dout), src, dst, edge_weight, znd)

    cb = 80
    nb1 = npad // cb
    out = pl.pallas_call(
        _comb_body,
        grid=(n // cb,),
        in_specs=[
            pl.BlockSpec((cb, dout), lambda i: (i, 0)),
            pl.BlockSpec((cb, dout), lambda i: (i, 0)),
            pl.BlockSpec((cb, dout), lambda i: (i + nb1, 0)),
        ],
        out_specs=pl.BlockSpec((cb, dout), lambda i: (i, 0)),
        out_shape=jax.ShapeDtypeStruct((n, dout), jnp.float32),
    )(base, partial, partial)
    return out


# speedup vs baseline: 36.6701x; 1.4504x over previous
import functools

import jax
import jax.numpy as jnp
from jax import lax
from jax.experimental import pallas as pl
from jax.experimental.pallas import tpu as pltpu
from jax.experimental.pallas import tpu_sc as plsc

NC = 2
NS = 16
CHUNK = 80


def _mm_body(f_ref, wt_ref, bsum_ref, base_ref, g_ref):
    f = f_ref[...]
    wt = wt_ref[...]
    din = f.shape[1]
    base_ref[...] = (
        jnp.dot(f, wt[0:din], preferred_element_type=jnp.float32) + bsum_ref[...]
    )
    g_ref[0] = jnp.dot(f, wt[din : 2 * din], preferred_element_type=jnp.float32)
    g_ref[1] = jnp.dot(f, wt[2 * din : 3 * din], preferred_element_type=jnp.float32)


def _comb_body(b_ref, p0_ref, p1_ref, o_ref):
    o_ref[...] = b_ref[...] + p0_ref[...] + p1_ref[...]


def _edge_vals(wv, dv, n):
    zf = jnp.zeros((16,), jnp.float32)
    zi = jnp.zeros((16,), jnp.int32)
    nvec = jnp.full((16,), n, jnp.int32)
    negv = wv < zf
    ni = jnp.where(negv, nvec, zi)
    ev = jnp.where(wv > zf, jnp.exp(wv), jnp.where(negv, jnp.exp(-wv), zf))
    return ev, dv + ni, ni


def _make_sc_kernel(n, e, d):
    ea = e // NS
    eb = e // (NC * NS)
    rpt = (-(-n // NS) + 39) // 40 * 40
    npad = NS * rpt
    sp = ((2 * n + NS * 16 - 1) // (NS * 16)) * NS * 16
    spt = sp // NS
    mesh = plsc.VectorSubcoreMesh(core_axis_name="c", subcore_axis_name="s")

    stage = 2000

    def body(g_hbm, src_hbm, dst_hbm, w_hbm, znd_hbm, out_hbm,
             ws, ds_, ss, ebuf, d2buf, zbuf, d2bufb, ebufb, denbuf,
             rows0, rows1, rows2, sidx0, sidx1, sidx2,
             dst0, dst1, dst2, scl0, scl1, scl2,
             gsem0, gsem1, gsem2, ssem0, ssem1, ssem2,
             s_tab, acc):
        c = lax.axis_index("c")
        s = lax.axis_index("s")
        wid = c * NS + s

        zv = jnp.zeros((16,), jnp.float32)

        def zero_body(i, _):
            zbuf[pl.ds(i * 16, 16)] = zv
            return 0

        lax.fori_loop(0, spt // 16, zero_body, 0)
        pltpu.sync_copy(zbuf, s_tab.at[pl.ds(s * spt, spt)])
        pltpu.sync_copy(znd_hbm.at[pl.ds(s * rpt, rpt)],
                        acc.at[pl.ds(s * rpt, rpt)])

        plsc.subcore_barrier()

        def chunk_a(i, _):
            off = i * CHUNK
            for j in range(CHUNK // 16):
                wv = ws[pl.ds(off + j * 16, 16)]
                dv = ds_[pl.ds(off + j * 16, 16)]
                ev, d2v, _ = _edge_vals(wv, dv, n)
                ebuf[pl.ds(j * 16, 16)] = ev
                d2buf[pl.ds(j * 16, 16)] = d2v
            pltpu.sync_copy(ebuf, s_tab.at[d2buf], add=True)
            return 0

        def stage_a(t, _):
            sbase = s * ea + t * stage
            pltpu.sync_copy(w_hbm.at[pl.ds(sbase, stage)], ws)
            pltpu.sync_copy(dst_hbm.at[pl.ds(sbase, stage)], ds_)
            lax.fori_loop(0, stage // CHUNK, chunk_a, 0)
            return 0

        lax.fori_loop(0, ea // stage, stage_a, 0)

        plsc.subcore_barrier()

        tiny = jnp.full((16,), 1e-30, jnp.float32)
        sets = ((rows0, sidx0, dst0, scl0, gsem0, ssem0),
                (rows1, sidx1, dst1, scl1, gsem1, ssem1),
                (rows2, sidx2, dst2, scl2, gsem2, ssem2))

        def prep(k, st):
            rowsx, sidxx, dstx, sclx, gsemx, _ = st
            off = k * CHUNK
            for j in range(CHUNK // 16):
                wv = ws[pl.ds(off + j * 16, 16)]
                dv = ds_[pl.ds(off + j * 16, 16)]
                srcv = ss[pl.ds(off + j * 16, 16)]
                ev, d2v, ni = _edge_vals(wv, dv, n)
                ebufb[pl.ds(j * 16, 16)] = ev
                d2bufb[pl.ds(j * 16, 16)] = d2v
                sidxx[pl.ds(j * 16, 16)] = srcv + ni
                dstx[pl.ds(j * 16, 16)] = dv
            pltpu.sync_copy(s_tab.at[d2bufb], denbuf)
            for j in range(CHUNK // 16):
                ev = ebufb[pl.ds(j * 16, 16)]
                denv = denbuf[pl.ds(j * 16, 16)]
                sclx[pl.ds(j * 16, 16)] = ev / jnp.maximum(denv, tiny)
            pltpu.async_copy(g_hbm.at[sidxx], rowsx, gsemx)

        def finish(st):
            rowsx, sidxx, dstx, sclx, gsemx, ssemx = st
            pltpu.make_async_copy(g_hbm.at[sidxx], rowsx, gsemx).wait()

            def srow(j, _):
                sva = sclx[pl.ds(j * 16, 16)]
                for l in range(16):
                    sv = lax.gather(
                        sva, jnp.full((16, 1), l, jnp.int32),
                        dimension_numbers=lax.GatherDimensionNumbers(
                            offset_dims=(), collapsed_slice_dims=(0,),
                            start_index_map=(0,)),
                        slice_sizes=(1,),
                        mode=lax.GatherScatterMode.PROMISE_IN_BOUNDS)
                    r = j * 16 + l
                    for t in range(d // 16):
                        rowsx[r, pl.ds(t * 16, 16)] = (
                            rowsx[r, pl.ds(t * 16, 16)] * sv)
                return 0

            lax.fori_loop(0, CHUNK // 16, srow, 0)
            pltpu.async_copy(rowsx, acc.at[dstx], ssemx, add=True)

        def wait_scatter(st):
            rowsx, _, dstx, _, _, ssemx = st
            pltpu.make_async_copy(rowsx, acc.at[dstx], ssemx).wait()

        def stage_b(t, _):
            sbase = wid * eb + t * stage
            pltpu.sync_copy(w_hbm.at[pl.ds(sbase, stage)], ws)
            pltpu.sync_copy(dst_hbm.at[pl.ds(sbase, stage)], ds_)
            pltpu.sync_copy(src_hbm.at[pl.ds(sbase, stage)], ss)

            prep(0, sets[0])

            def rot(m, _):
                for i in range(3):
                    if i == 2:
                        wait_scatter(sets[0])
                    else:

                        @pl.when(m > 0)
                        def _():
                            wait_scatter(sets[i + 1])

                    prep(3 * m + 1 + i, sets[(i + 1) % 3])
                    finish(sets[i % 3])
                return 0

            nrot = (stage // CHUNK - 1) // 3
            lax.fori_loop(0, nrot, rot, 0)
            finish(sets[0])
            for st in sets:
                wait_scatter(st)
            return 0

        lax.fori_loop(0, eb // stage, stage_b, 0)

        plsc.subcore_barrier()
        pltpu.sync_copy(acc.at[pl.ds(s * rpt, rpt)],
                        out_hbm.at[pl.ds(c * npad + s * rpt, rpt)])

    return pl.kernel(
        body,
        out_type=jax.ShapeDtypeStruct((NC * npad, d), jnp.float32),
        mesh=mesh,
        scratch_types=[
            pltpu.VMEM((stage,), jnp.float32),
            pltpu.VMEM((stage,), jnp.int32),
            pltpu.VMEM((stage,), jnp.int32),
            pltpu.VMEM((CHUNK,), jnp.float32),
            pltpu.VMEM((CHUNK,), jnp.int32),
            pltpu.VMEM((spt,), jnp.float32),
            pltpu.VMEM((CHUNK,), jnp.int32),
            pltpu.VMEM((CHUNK,), jnp.float32),
            pltpu.VMEM((CHUNK,), jnp.float32),
            pltpu.VMEM((CHUNK, d), jnp.float32),
            pltpu.VMEM((CHUNK, d), jnp.float32),
            pltpu.VMEM((CHUNK, d), jnp.float32),
            pltpu.VMEM((CHUNK,), jnp.int32),
            pltpu.VMEM((CHUNK,), jnp.int32),
            pltpu.VMEM((CHUNK,), jnp.int32),
            pltpu.VMEM((CHUNK,), jnp.int32),
            pltpu.VMEM((CHUNK,), jnp.int32),
            pltpu.VMEM((CHUNK,), jnp.int32),
            pltpu.VMEM((CHUNK,), jnp.float32),
            pltpu.VMEM((CHUNK,), jnp.float32),
            pltpu.VMEM((CHUNK,), jnp.float32),
            pltpu.SemaphoreType.DMA,
            pltpu.SemaphoreType.DMA,
            pltpu.SemaphoreType.DMA,
            pltpu.SemaphoreType.DMA,
            pltpu.SemaphoreType.DMA,
            pltpu.SemaphoreType.DMA,
            pltpu.VMEM_SHARED((sp,), jnp.float32),
            pltpu.VMEM_SHARED((npad, d), jnp.float32),
        ],
    )


def kernel(feat, edge_index, edge_weight, W, b_fc, bias,
           coef_self, coef_posi, coef_nega):
    n, din = feat.shape
    dout = W.shape[0]
    e = edge_weight.shape[0]

    wt = W.T
    wts = jnp.concatenate(
        [wt[:din] * coef_self, wt[din:2 * din] * coef_posi,
         wt[2 * din:] * coef_nega], axis=0)
    bsum = (b_fc + bias).reshape(1, dout)

    rb = 1000
    grid = (n // rb,)
    base, g = pl.pallas_call(
        _mm_body,
        grid=grid,
        in_specs=[
            pl.BlockSpec((rb, din), lambda i: (i, 0)),
            pl.BlockSpec((3 * din, dout), lambda i: (0, 0)),
            pl.BlockSpec((1, dout), lambda i: (0, 0)),
        ],
        out_specs=[
            pl.BlockSpec((rb, dout), lambda i: (i, 0)),
            pl.BlockSpec((2, rb, dout), lambda i: (0, i, 0)),
        ],
        out_shape=[
            jax.ShapeDtypeStruct((n, dout), jnp.float32),
            jax.ShapeDtypeStruct((2, n, dout), jnp.float32),
        ],
    )(feat, wts, bsum)

    src = edge_index[0]
    dst = edge_index[1]
    rpt = (-(-n // NS) + 39) // 40 * 40
    npad = NS * rpt
    znd = jnp.zeros((npad, dout), jnp.float32)
    sc = _make_sc_kernel(n, e, dout)
    partial = sc(g.reshape(2 * n, dout), src, dst, edge_weight, znd)

    cb = 80
    nb1 = npad // cb
    out = pl.pallas_call(
        _comb_body,
        grid=(n // cb,),
        in_specs=[
            pl.BlockSpec((cb, dout), lambda i: (i, 0)),
            pl.BlockSpec((cb, dout), lambda i: (i, 0)),
            pl.BlockSpec((cb, dout), lambda i: (i + nb1, 0)),
        ],
        out_specs=pl.BlockSpec((cb, dout), lambda i: (i, 0)),
        out_shape=jax.ShapeDtypeStruct((n, dout), jnp.float32),
    )(base, partial, partial)
    return out


# speedup vs baseline: 39.6308x; 1.0807x over previous
import functools

import jax
import jax.numpy as jnp
from jax import lax
from jax.experimental import pallas as pl
from jax.experimental.pallas import tpu as pltpu
from jax.experimental.pallas import tpu_sc as plsc

NC = 2
NS = 16
CHUNK = 80


def _mm_body(f_ref, wt_ref, bsum_ref, base_ref, g_ref):
    f = f_ref[...]
    wt = wt_ref[...]
    din = f.shape[1]
    base_ref[...] = (
        jnp.dot(f, wt[0:din], preferred_element_type=jnp.float32) + bsum_ref[...]
    )
    g_ref[0] = jnp.dot(f, wt[din : 2 * din], preferred_element_type=jnp.float32)
    g_ref[1] = jnp.dot(f, wt[2 * din : 3 * din], preferred_element_type=jnp.float32)


def _comb_body(b_ref, p0_ref, p1_ref, o_ref):
    o_ref[...] = b_ref[...] + p0_ref[...] + p1_ref[...]


def _edge_vals(wv, dv, n):
    zf = jnp.zeros((16,), jnp.float32)
    zi = jnp.zeros((16,), jnp.int32)
    nvec = jnp.full((16,), n, jnp.int32)
    negv = wv < zf
    ni = jnp.where(negv, nvec, zi)
    ev = jnp.where(wv > zf, jnp.exp(wv), jnp.where(negv, jnp.exp(-wv), zf))
    return ev, dv + ni, ni


def _make_sc_kernel(n, e, d):
    ea = e // NS
    eb = e // (NC * NS)
    rpt = (-(-n // NS) + 39) // 40 * 40
    npad = NS * rpt
    sp = ((2 * n + NS * 16 - 1) // (NS * 16)) * NS * 16
    spt = sp // NS
    mesh = plsc.VectorSubcoreMesh(core_axis_name="c", subcore_axis_name="s")

    stage = 2000

    def body(g_hbm, src_hbm, dst_hbm, w_hbm, znd_hbm, out_hbm,
             ws, ds_, ss, zbuf,
             rows0, rows1, rows2, sidx0, sidx1, sidx2,
             dst0, dst1, dst2, e0, e1, e2, d20, d21, d22,
             den0, den1, den2,
             gsem0, gsem1, gsem2, ssem0, ssem1, ssem2,
             dsem0, dsem1, dsem2,
             s_tab, acc):
        c = lax.axis_index("c")
        s = lax.axis_index("s")
        wid = c * NS + s

        zv = jnp.zeros((16,), jnp.float32)

        def zero_body(i, _):
            zbuf[pl.ds(i * 16, 16)] = zv
            return 0

        lax.fori_loop(0, spt // 16, zero_body, 0)
        pltpu.sync_copy(zbuf, s_tab.at[pl.ds(s * spt, spt)])
        pltpu.sync_copy(znd_hbm.at[pl.ds(s * rpt, rpt)],
                        acc.at[pl.ds(s * rpt, rpt)])

        plsc.subcore_barrier()

        def comp_a(i, ebx, d2x):
            off = i * CHUNK
            for j in range(CHUNK // 16):
                wv = ws[pl.ds(off + j * 16, 16)]
                dv = ds_[pl.ds(off + j * 16, 16)]
                ev, d2v, _ = _edge_vals(wv, dv, n)
                ebx[pl.ds(j * 16, 16)] = ev
                d2x[pl.ds(j * 16, 16)] = d2v

        def issue_a(ebx, d2x, semx):
            pltpu.async_copy(ebx, s_tab.at[d2x], semx, add=True)

        def wait_a(ebx, d2x, semx):
            pltpu.make_async_copy(ebx, s_tab.at[d2x], semx).wait()

        def stage_a(t, _):
            sbase = s * ea + t * stage
            pltpu.sync_copy(w_hbm.at[pl.ds(sbase, stage)], ws)
            pltpu.sync_copy(dst_hbm.at[pl.ds(sbase, stage)], ds_)
            comp_a(0, e0, d20)
            issue_a(e0, d20, dsem0)

            def dbl(m, _):
                @pl.when(m > 0)
                def _():
                    wait_a(e1, d21, dsem1)

                comp_a(2 * m + 1, e1, d21)
                issue_a(e1, d21, dsem1)
                wait_a(e0, d20, dsem0)
                comp_a(2 * m + 2, e0, d20)
                issue_a(e0, d20, dsem0)
                return 0

            lax.fori_loop(0, (stage // CHUNK - 1) // 2, dbl, 0)
            wait_a(e0, d20, dsem0)
            wait_a(e1, d21, dsem1)
            return 0

        lax.fori_loop(0, ea // stage, stage_a, 0)

        plsc.subcore_barrier()

        tiny = jnp.full((16,), 1e-30, jnp.float32)
        sets = ((rows0, sidx0, dst0, e0, d20, den0, gsem0, ssem0, dsem0),
                (rows1, sidx1, dst1, e1, d21, den1, gsem1, ssem1, dsem1),
                (rows2, sidx2, dst2, e2, d22, den2, gsem2, ssem2, dsem2))

        def prep(k, st):
            rowsx, sidxx, dstx, ex, d2x, denx, gsemx, _, dsemx = st
            off = k * CHUNK
            for j in range(CHUNK // 16):
                wv = ws[pl.ds(off + j * 16, 16)]
                dv = ds_[pl.ds(off + j * 16, 16)]
                srcv = ss[pl.ds(off + j * 16, 16)]
                ev, d2v, ni = _edge_vals(wv, dv, n)
                ex[pl.ds(j * 16, 16)] = ev
                d2x[pl.ds(j * 16, 16)] = d2v
                sidxx[pl.ds(j * 16, 16)] = srcv + ni
                dstx[pl.ds(j * 16, 16)] = dv
            pltpu.async_copy(s_tab.at[d2x], denx, dsemx)
            pltpu.async_copy(g_hbm.at[sidxx], rowsx, gsemx)

        def finish(st):
            rowsx, sidxx, dstx, ex, d2x, denx, gsemx, ssemx, dsemx = st
            pltpu.make_async_copy(s_tab.at[d2x], denx, dsemx).wait()
            pltpu.make_async_copy(g_hbm.at[sidxx], rowsx, gsemx).wait()

            def srow(j, _):
                ev16 = ex[pl.ds(j * 16, 16)]
                dn16 = denx[pl.ds(j * 16, 16)]
                sva = ev16 / jnp.maximum(dn16, tiny)
                for l in range(16):
                    sv = lax.gather(
                        sva, jnp.full((16, 1), l, jnp.int32),
                        dimension_numbers=lax.GatherDimensionNumbers(
                            offset_dims=(), collapsed_slice_dims=(0,),
                            start_index_map=(0,)),
                        slice_sizes=(1,),
                        mode=lax.GatherScatterMode.PROMISE_IN_BOUNDS)
                    r = j * 16 + l
                    for t in range(d // 16):
                        rowsx[r, pl.ds(t * 16, 16)] = (
                            rowsx[r, pl.ds(t * 16, 16)] * sv)
                return 0

            lax.fori_loop(0, CHUNK // 16, srow, 0)
            pltpu.async_copy(rowsx, acc.at[dstx], ssemx, add=True)

        def wait_scatter(st):
            rowsx, _, dstx = st[0], st[1], st[2]
            ssemx = st[7]
            pltpu.make_async_copy(rowsx, acc.at[dstx], ssemx).wait()

        def stage_b(t, _):
            sbase = wid * eb + t * stage
            pltpu.sync_copy(w_hbm.at[pl.ds(sbase, stage)], ws)
            pltpu.sync_copy(dst_hbm.at[pl.ds(sbase, stage)], ds_)
            pltpu.sync_copy(src_hbm.at[pl.ds(sbase, stage)], ss)

            prep(0, sets[0])

            def rot(m, _):
                for i in range(3):
                    if i == 2:
                        wait_scatter(sets[0])
                    else:

                        @pl.when(m > 0)
                        def _():
                            wait_scatter(sets[i + 1])

                    prep(3 * m + 1 + i, sets[(i + 1) % 3])
                    finish(sets[i % 3])
                return 0

            nrot = (stage // CHUNK - 1) // 3
            lax.fori_loop(0, nrot, rot, 0)
            finish(sets[0])
            for st in sets:
                wait_scatter(st)
            return 0

        lax.fori_loop(0, eb // stage, stage_b, 0)

        plsc.subcore_barrier()
        pltpu.sync_copy(acc.at[pl.ds(s * rpt, rpt)],
                        out_hbm.at[pl.ds(c * npad + s * rpt, rpt)])

    return pl.kernel(
        body,
        out_type=jax.ShapeDtypeStruct((NC * npad, d), jnp.float32),
        mesh=mesh,
        scratch_types=[
            pltpu.VMEM((stage,), jnp.float32),
            pltpu.VMEM((stage,), jnp.int32),
            pltpu.VMEM((stage,), jnp.int32),
            pltpu.VMEM((spt,), jnp.float32),
            pltpu.VMEM((CHUNK, d), jnp.float32),
            pltpu.VMEM((CHUNK, d), jnp.float32),
            pltpu.VMEM((CHUNK, d), jnp.float32),
            pltpu.VMEM((CHUNK,), jnp.int32),
            pltpu.VMEM((CHUNK,), jnp.int32),
            pltpu.VMEM((CHUNK,), jnp.int32),
            pltpu.VMEM((CHUNK,), jnp.int32),
            pltpu.VMEM((CHUNK,), jnp.int32),
            pltpu.VMEM((CHUNK,), jnp.int32),
            pltpu.VMEM((CHUNK,), jnp.float32),
            pltpu.VMEM((CHUNK,), jnp.float32),
            pltpu.VMEM((CHUNK,), jnp.float32),
            pltpu.VMEM((CHUNK,), jnp.int32),
            pltpu.VMEM((CHUNK,), jnp.int32),
            pltpu.VMEM((CHUNK,), jnp.int32),
            pltpu.VMEM((CHUNK,), jnp.float32),
            pltpu.VMEM((CHUNK,), jnp.float32),
            pltpu.VMEM((CHUNK,), jnp.float32),
            pltpu.SemaphoreType.DMA,
            pltpu.SemaphoreType.DMA,
            pltpu.SemaphoreType.DMA,
            pltpu.SemaphoreType.DMA,
            pltpu.SemaphoreType.DMA,
            pltpu.SemaphoreType.DMA,
            pltpu.SemaphoreType.DMA,
            pltpu.SemaphoreType.DMA,
            pltpu.SemaphoreType.DMA,
            pltpu.VMEM_SHARED((sp,), jnp.float32),
            pltpu.VMEM_SHARED((npad, d), jnp.float32),
        ],
    )


def kernel(feat, edge_index, edge_weight, W, b_fc, bias,
           coef_self, coef_posi, coef_nega):
    n, din = feat.shape
    dout = W.shape[0]
    e = edge_weight.shape[0]

    wt = W.T
    wts = jnp.concatenate(
        [wt[:din] * coef_self, wt[din:2 * din] * coef_posi,
         wt[2 * din:] * coef_nega], axis=0)
    bsum = (b_fc + bias).reshape(1, dout)

    rb = 1000
    grid = (n // rb,)
    base, g = pl.pallas_call(
        _mm_body,
        grid=grid,
        in_specs=[
            pl.BlockSpec((rb, din), lambda i: (i, 0)),
            pl.BlockSpec((3 * din, dout), lambda i: (0, 0)),
            pl.BlockSpec((1, dout), lambda i: (0, 0)),
        ],
        out_specs=[
            pl.BlockSpec((rb, dout), lambda i: (i, 0)),
            pl.BlockSpec((2, rb, dout), lambda i: (0, i, 0)),
        ],
        out_shape=[
            jax.ShapeDtypeStruct((n, dout), jnp.float32),
            jax.ShapeDtypeStruct((2, n, dout), jnp.float32),
        ],
    )(feat, wts, bsum)

    src = edge_index[0]
    dst = edge_index[1]
    rpt = (-(-n // NS) + 39) // 40 * 40
    npad = NS * rpt
    znd = jnp.zeros((npad, dout), jnp.float32)
    sc = _make_sc_kernel(n, e, dout)
    partial = sc(g.reshape(2 * n, dout), src, dst, edge_weight, znd)

    cb = 80
    nb1 = npad // cb
    out = pl.pallas_call(
        _comb_body,
        grid=(n // cb,),
        in_specs=[
            pl.BlockSpec((cb, dout), lambda i: (i, 0)),
            pl.BlockSpec((cb, dout), lambda i: (i, 0)),
            pl.BlockSpec((cb, dout), lambda i: (i + nb1, 0)),
        ],
        out_specs=pl.BlockSpec((cb, dout), lambda i: (i, 0)),
        out_shape=jax.ShapeDtypeStruct((n, dout), jnp.float32),
    )(base, partial, partial)
    return out


# speedup vs baseline: 49.6674x; 1.2533x over previous
import functools

import jax
import jax.numpy as jnp
from jax import lax
from jax.experimental import pallas as pl
from jax.experimental.pallas import tpu as pltpu
from jax.experimental.pallas import tpu_sc as plsc

NC = 2
NS = 16
CHUNK = 80


def _mm_body(f_ref, wt_ref, b1_ref, b2_ref, o_ref):
    h = pl.program_id(0)
    f = f_ref[...]
    o_ref[...] = jnp.dot(f, wt_ref[0], preferred_element_type=jnp.float32)

    @pl.when(h == 0)
    def _():
        o_ref[...] += b1_ref[...] + b2_ref[...]


def _comb_body(b_ref, p0_ref, p1_ref, o_ref):
    o_ref[...] = b_ref[...] + p0_ref[...] + p1_ref[...]


def _edge_vals(wv, dv, n):
    zf = jnp.zeros((16,), jnp.float32)
    zi = jnp.zeros((16,), jnp.int32)
    nvec = jnp.full((16,), n, jnp.int32)
    negv = wv < zf
    ni = jnp.where(negv, nvec, zi)
    ev = jnp.where(wv > zf, jnp.exp(wv), jnp.where(negv, jnp.exp(-wv), zf))
    return ev, dv + ni, ni


def _make_sc_kernel(n, e, d):
    ea = e // NS
    eb = e // (NC * NS)
    rpt = (-(-n // NS) + 39) // 40 * 40
    npad = NS * rpt
    sp = ((2 * n + NS * 16 - 1) // (NS * 16)) * NS * 16
    spt = sp // NS
    mesh = plsc.VectorSubcoreMesh(core_axis_name="c", subcore_axis_name="s")

    stage = 2000

    def body(g_hbm, ei_hbm, w_hbm, out0_hbm, out1_hbm,
             ws, ds_, ss, zbuf,
             rows0, rows1, rows2, sidx0, sidx1, sidx2,
             dst0, dst1, dst2, e0, e1, e2, d20, d21, d22,
             den0, den1, den2,
             gsem0, gsem1, gsem2, ssem0, ssem1, ssem2,
             dsem0, dsem1, dsem2,
             s_tab, acc):
        c = lax.axis_index("c")
        s = lax.axis_index("s")
        wid = c * NS + s

        zv = jnp.zeros((16,), jnp.float32)

        def zero_body(i, _):
            zbuf[pl.ds(i * 16, 16)] = zv
            return 0

        lax.fori_loop(0, spt // 16, zero_body, 0)
        pltpu.sync_copy(zbuf, s_tab.at[pl.ds(s * spt, spt)])

        def zrow(i, _):
            for t in range(d // 16):
                rows0[i, pl.ds(t * 16, 16)] = zv
            return 0

        lax.fori_loop(0, CHUNK, zrow, 0)
        for k in range(rpt // CHUNK):
            pltpu.sync_copy(rows0,
                            acc.at[pl.ds(s * rpt + k * CHUNK, CHUNK)])

        plsc.subcore_barrier()

        def comp_a(i, ebx, d2x):
            off = i * CHUNK
            for j in range(CHUNK // 16):
                wv = ws[pl.ds(off + j * 16, 16)]
                dv = ds_[pl.ds(off + j * 16, 16)]
                ev, d2v, _ = _edge_vals(wv, dv, n)
                ebx[pl.ds(j * 16, 16)] = ev
                d2x[pl.ds(j * 16, 16)] = d2v

        def issue_a(ebx, d2x, semx):
            pltpu.async_copy(ebx, s_tab.at[d2x], semx, add=True)

        def wait_a(ebx, d2x, semx):
            pltpu.make_async_copy(ebx, s_tab.at[d2x], semx).wait()

        def stage_a(t, _):
            sbase = s * ea + t * stage
            pltpu.sync_copy(w_hbm.at[pl.ds(sbase, stage)], ws)
            pltpu.sync_copy(ei_hbm.at[pl.ds(e + sbase, stage)], ds_)
            comp_a(0, e0, d20)
            issue_a(e0, d20, dsem0)

            def dbl(m, _):
                @pl.when(m > 0)
                def _():
                    wait_a(e1, d21, dsem1)

                comp_a(2 * m + 1, e1, d21)
                issue_a(e1, d21, dsem1)
                wait_a(e0, d20, dsem0)
                comp_a(2 * m + 2, e0, d20)
                issue_a(e0, d20, dsem0)
                return 0

            lax.fori_loop(0, (stage // CHUNK - 1) // 2, dbl, 0)
            wait_a(e0, d20, dsem0)
            wait_a(e1, d21, dsem1)
            return 0

        lax.fori_loop(0, ea // stage, stage_a, 0)

        plsc.subcore_barrier()

        tiny = jnp.full((16,), 1e-30, jnp.float32)
        sets = ((rows0, sidx0, dst0, e0, d20, den0, gsem0, ssem0, dsem0),
                (rows1, sidx1, dst1, e1, d21, den1, gsem1, ssem1, dsem1),
                (rows2, sidx2, dst2, e2, d22, den2, gsem2, ssem2, dsem2))

        def prep(k, st):
            rowsx, sidxx, dstx, ex, d2x, denx, gsemx, _, dsemx = st
            off = k * CHUNK
            for j in range(CHUNK // 16):
                wv = ws[pl.ds(off + j * 16, 16)]
                dv = ds_[pl.ds(off + j * 16, 16)]
                srcv = ss[pl.ds(off + j * 16, 16)]
                ev, d2v, ni = _edge_vals(wv, dv, n)
                ex[pl.ds(j * 16, 16)] = ev
                d2x[pl.ds(j * 16, 16)] = d2v
                sidxx[pl.ds(j * 16, 16)] = (
                    srcv + ni + jnp.full((16,), n, jnp.int32))
                dstx[pl.ds(j * 16, 16)] = dv
            pltpu.async_copy(s_tab.at[d2x], denx, dsemx)
            pltpu.async_copy(g_hbm.at[sidxx], rowsx, gsemx)

        def finish(st):
            rowsx, sidxx, dstx, ex, d2x, denx, gsemx, ssemx, dsemx = st
            pltpu.make_async_copy(s_tab.at[d2x], denx, dsemx).wait()
            pltpu.make_async_copy(g_hbm.at[sidxx], rowsx, gsemx).wait()

            def srow(j, _):
                ev16 = ex[pl.ds(j * 16, 16)]
                dn16 = denx[pl.ds(j * 16, 16)]
                sva = ev16 / jnp.maximum(dn16, tiny)
                for l in range(16):
                    sv = lax.gather(
                        sva, jnp.full((16, 1), l, jnp.int32),
                        dimension_numbers=lax.GatherDimensionNumbers(
                            offset_dims=(), collapsed_slice_dims=(0,),
                            start_index_map=(0,)),
                        slice_sizes=(1,),
                        mode=lax.GatherScatterMode.PROMISE_IN_BOUNDS)
                    r = j * 16 + l
                    for t in range(d // 16):
                        rowsx[r, pl.ds(t * 16, 16)] = (
                            rowsx[r, pl.ds(t * 16, 16)] * sv)
                return 0

            lax.fori_loop(0, CHUNK // 16, srow, 0)
            pltpu.async_copy(rowsx, acc.at[dstx], ssemx, add=True)

        def wait_scatter(st):
            rowsx, _, dstx = st[0], st[1], st[2]
            ssemx = st[7]
            pltpu.make_async_copy(rowsx, acc.at[dstx], ssemx).wait()

        def stage_b(t, _):
            sbase = wid * eb + t * stage
            pltpu.sync_copy(w_hbm.at[pl.ds(sbase, stage)], ws)
            pltpu.sync_copy(ei_hbm.at[pl.ds(e + sbase, stage)], ds_)
            pltpu.sync_copy(ei_hbm.at[pl.ds(sbase, stage)], ss)

            prep(0, sets[0])

            def rot(m, _):
                for i in range(3):
                    if i == 2:
                        wait_scatter(sets[0])
                    else:

                        @pl.when(m > 0)
                        def _():
                            wait_scatter(sets[i + 1])

                    prep(3 * m + 1 + i, sets[(i + 1) % 3])
                    finish(sets[i % 3])
                return 0

            nrot = (stage // CHUNK - 1) // 3
            lax.fori_loop(0, nrot, rot, 0)
            finish(sets[0])
            for st in sets:
                wait_scatter(st)
            return 0

        lax.fori_loop(0, eb // stage, stage_b, 0)

        plsc.subcore_barrier()

        @pl.when(c == 0)
        def _():
            pltpu.sync_copy(acc.at[pl.ds(s * rpt, rpt)],
                            out0_hbm.at[pl.ds(s * rpt, rpt)])

        @pl.when(c == 1)
        def _():
            pltpu.sync_copy(acc.at[pl.ds(s * rpt, rpt)],
                            out1_hbm.at[pl.ds(s * rpt, rpt)])

    return pl.kernel(
        body,
        out_type=[jax.ShapeDtypeStruct((npad, d), jnp.float32),
                  jax.ShapeDtypeStruct((npad, d), jnp.float32)],
        mesh=mesh,
        scratch_types=[
            pltpu.VMEM((stage,), jnp.float32),
            pltpu.VMEM((stage,), jnp.int32),
            pltpu.VMEM((stage,), jnp.int32),
            pltpu.VMEM((spt,), jnp.float32),
            pltpu.VMEM((CHUNK, d), jnp.float32),
            pltpu.VMEM((CHUNK, d), jnp.float32),
            pltpu.VMEM((CHUNK, d), jnp.float32),
            pltpu.VMEM((CHUNK,), jnp.int32),
            pltpu.VMEM((CHUNK,), jnp.int32),
            pltpu.VMEM((CHUNK,), jnp.int32),
            pltpu.VMEM((CHUNK,), jnp.int32),
            pltpu.VMEM((CHUNK,), jnp.int32),
            pltpu.VMEM((CHUNK,), jnp.int32),
            pltpu.VMEM((CHUNK,), jnp.float32),
            pltpu.VMEM((CHUNK,), jnp.float32),
            pltpu.VMEM((CHUNK,), jnp.float32),
            pltpu.VMEM((CHUNK,), jnp.int32),
            pltpu.VMEM((CHUNK,), jnp.int32),
            pltpu.VMEM((CHUNK,), jnp.int32),
            pltpu.VMEM((CHUNK,), jnp.float32),
            pltpu.VMEM((CHUNK,), jnp.float32),
            pltpu.VMEM((CHUNK,), jnp.float32),
            pltpu.SemaphoreType.DMA,
            pltpu.SemaphoreType.DMA,
            pltpu.SemaphoreType.DMA,
            pltpu.SemaphoreType.DMA,
            pltpu.SemaphoreType.DMA,
            pltpu.SemaphoreType.DMA,
            pltpu.SemaphoreType.DMA,
            pltpu.SemaphoreType.DMA,
            pltpu.SemaphoreType.DMA,
            pltpu.VMEM_SHARED((sp,), jnp.float32),
            pltpu.VMEM_SHARED((npad, d), jnp.float32),
        ],
    )


def kernel(feat, edge_index, edge_weight, W, b_fc, bias,
           coef_self, coef_posi, coef_nega):
    n, din = feat.shape
    dout = W.shape[0]
    e = edge_weight.shape[0]

    wt = W.T
    wts = jnp.concatenate(
        [wt[:din] * coef_self, wt[din:2 * din] * coef_posi,
         wt[2 * din:] * coef_nega], axis=0).reshape(3, din, dout)

    rb = 1000
    mm = pl.pallas_call(
        _mm_body,
        grid=(3, n // rb),
        in_specs=[
            pl.BlockSpec((rb, din), lambda h, i: (i, 0)),
            pl.BlockSpec((1, din, dout), lambda h, i: (h, 0, 0)),
            pl.BlockSpec((1, dout), lambda h, i: (0, 0)),
            pl.BlockSpec((1, dout), lambda h, i: (0, 0)),
        ],
        out_specs=pl.BlockSpec((rb, dout), lambda h, i: (h * (n // rb) + i, 0)),
        out_shape=jax.ShapeDtypeStruct((3 * n, dout), jnp.float32),
    )(feat, wts, b_fc.reshape(1, dout), bias.reshape(1, dout))

    sc = _make_sc_kernel(n, e, dout)
    p0, p1 = sc(mm, edge_index.reshape(2 * e), edge_weight)

    cb = 1000
    out = pl.pallas_call(
        _comb_body,
        grid=(n // cb,),
        in_specs=[
            pl.BlockSpec((cb, dout), lambda i: (i, 0)),
            pl.BlockSpec((cb, dout), lambda i: (i, 0)),
            pl.BlockSpec((cb, dout), lambda i: (i, 0)),
        ],
        out_specs=pl.BlockSpec((cb, dout), lambda i: (i, 0)),
        out_shape=jax.ShapeDtypeStruct((n, dout), jnp.float32),
    )(mm, p0, p1)
    return out


# speedup vs baseline: 51.6784x; 1.0405x over previous
import functools

import jax
import jax.numpy as jnp
from jax import lax
from jax.experimental import pallas as pl
from jax.experimental.pallas import tpu as pltpu
from jax.experimental.pallas import tpu_sc as plsc

NC = 2
NS = 16
CHUNK = 80


def _mm_body(f_ref, wt_ref, b1_ref, b2_ref, o_ref):
    h = pl.program_id(0)
    f = f_ref[...]
    o_ref[...] = jnp.dot(f, wt_ref[0], preferred_element_type=jnp.float32)

    @pl.when(h == 0)
    def _():
        o_ref[...] += b1_ref[...] + b2_ref[...]


def _comb_body(b_ref, p0_ref, p1_ref, o_ref):
    o_ref[...] = b_ref[...] + p0_ref[...] + p1_ref[...]


def _edge_vals(wv, dv, n):
    zf = jnp.zeros((16,), jnp.float32)
    zi = jnp.zeros((16,), jnp.int32)
    nvec = jnp.full((16,), n, jnp.int32)
    negv = wv < zf
    ni = jnp.where(negv, nvec, zi)
    ev = jnp.where(wv > zf, jnp.exp(wv), jnp.where(negv, jnp.exp(-wv), zf))
    return ev, dv + ni, ni


def _make_sc_kernel(n, e, d):
    ea = e // NS
    eb = e // (NC * NS)
    rpt = (-(-n // NS) + 39) // 40 * 40
    npad = NS * rpt
    sp = ((2 * n + NS * 16 - 1) // (NS * 16)) * NS * 16
    spt = sp // NS
    mesh = plsc.VectorSubcoreMesh(core_axis_name="c", subcore_axis_name="s")

    stage = 2000

    def body(g_hbm, ei_hbm, w_hbm, out0_hbm, out1_hbm,
             ws, ds_, ss, zbuf,
             rows0, rows1, rows2, sidx0, sidx1, sidx2,
             dst0, dst1, dst2, e0, e1, e2, d20, d21, d22,
             den0, den1, den2,
             gsem0, gsem1, gsem2, ssem0, ssem1, ssem2,
             dsem0, dsem1, dsem2,
             s_tab, acc):
        c = lax.axis_index("c")
        s = lax.axis_index("s")
        wid = c * NS + s

        zv = jnp.zeros((16,), jnp.float32)

        def zero_body(i, _):
            zbuf[pl.ds(i * 16, 16)] = zv
            return 0

        lax.fori_loop(0, spt // 16, zero_body, 0)
        pltpu.sync_copy(zbuf, s_tab.at[pl.ds(s * spt, spt)])

        def zrow(i, _):
            for t in range(d // 16):
                rows0[i, pl.ds(t * 16, 16)] = zv
            return 0

        lax.fori_loop(0, CHUNK, zrow, 0)
        for k in range(rpt // CHUNK):
            pltpu.sync_copy(rows0,
                            acc.at[pl.ds(s * rpt + k * CHUNK, CHUNK)])

        plsc.subcore_barrier()

        def comp_a(i, ebx, d2x):
            off = i * CHUNK
            for j in range(CHUNK // 16):
                wv = ws[pl.ds(off + j * 16, 16)]
                dv = ds_[pl.ds(off + j * 16, 16)]
                ev, d2v, _ = _edge_vals(wv, dv, n)
                ebx[pl.ds(j * 16, 16)] = ev
                d2x[pl.ds(j * 16, 16)] = d2v

        def issue_a(ebx, d2x, semx):
            pltpu.async_copy(ebx, s_tab.at[d2x], semx, add=True)

        def wait_a(ebx, d2x, semx):
            pltpu.make_async_copy(ebx, s_tab.at[d2x], semx).wait()

        def stage_a(t, _):
            sbase = s * ea + t * stage
            pltpu.sync_copy(w_hbm.at[pl.ds(sbase, stage)], ws)
            pltpu.sync_copy(ei_hbm.at[pl.ds(e + sbase, stage)], ds_)
            comp_a(0, e0, d20)
            issue_a(e0, d20, dsem0)

            def dbl(m, _):
                @pl.when(m > 0)
                def _():
                    wait_a(e1, d21, dsem1)

                comp_a(2 * m + 1, e1, d21)
                issue_a(e1, d21, dsem1)
                wait_a(e0, d20, dsem0)
                comp_a(2 * m + 2, e0, d20)
                issue_a(e0, d20, dsem0)
                return 0

            lax.fori_loop(0, (stage // CHUNK - 1) // 2, dbl, 0)
            wait_a(e0, d20, dsem0)
            wait_a(e1, d21, dsem1)
            return 0

        lax.fori_loop(0, ea // stage, stage_a, 0)

        plsc.subcore_barrier()

        tiny = jnp.full((16,), 1e-30, jnp.float32)
        sets = ((rows0, sidx0, dst0, e0, d20, den0, gsem0, ssem0, dsem0),
                (rows1, sidx1, dst1, e1, d21, den1, gsem1, ssem1, dsem1),
                (rows2, sidx2, dst2, e2, d22, den2, gsem2, ssem2, dsem2))

        def prep(k, st):
            rowsx, sidxx, dstx, ex, d2x, denx, gsemx, _, dsemx = st
            off = k * CHUNK
            for j in range(CHUNK // 16):
                wv = ws[pl.ds(off + j * 16, 16)]
                dv = ds_[pl.ds(off + j * 16, 16)]
                srcv = ss[pl.ds(off + j * 16, 16)]
                ev, d2v, ni = _edge_vals(wv, dv, n)
                ex[pl.ds(j * 16, 16)] = ev
                d2x[pl.ds(j * 16, 16)] = d2v
                sidxx[pl.ds(j * 16, 16)] = (
                    srcv + ni + jnp.full((16,), n, jnp.int32))
                dstx[pl.ds(j * 16, 16)] = dv
            pltpu.async_copy(s_tab.at[d2x], denx, dsemx)
            pltpu.async_copy(g_hbm.at[sidxx], rowsx, gsemx)

        def finish(st):
            rowsx, sidxx, dstx, ex, d2x, denx, gsemx, ssemx, dsemx = st
            pltpu.make_async_copy(s_tab.at[d2x], denx, dsemx).wait()
            pltpu.make_async_copy(g_hbm.at[sidxx], rowsx, gsemx).wait()

            def srow(j, _):
                ev16 = ex[pl.ds(j * 16, 16)]
                dn16 = denx[pl.ds(j * 16, 16)]
                sva = ev16 / jnp.maximum(dn16, tiny)
                for l in range(16):
                    sv = lax.gather(
                        sva, jnp.full((16, 1), l, jnp.int32),
                        dimension_numbers=lax.GatherDimensionNumbers(
                            offset_dims=(), collapsed_slice_dims=(0,),
                            start_index_map=(0,)),
                        slice_sizes=(1,),
                        mode=lax.GatherScatterMode.PROMISE_IN_BOUNDS)
                    r = j * 16 + l
                    for t in range(d // 16):
                        rowsx[r, pl.ds(t * 16, 16)] = (
                            rowsx[r, pl.ds(t * 16, 16)] * sv)
                return 0

            lax.fori_loop(0, CHUNK // 16, srow, 0)
            pltpu.async_copy(rowsx, acc.at[dstx], ssemx, add=True)

        def wait_scatter(st):
            rowsx, _, dstx = st[0], st[1], st[2]
            ssemx = st[7]
            pltpu.make_async_copy(rowsx, acc.at[dstx], ssemx).wait()

        def stage_b(t, _):
            sbase = wid * eb + t * stage
            pltpu.sync_copy(w_hbm.at[pl.ds(sbase, stage)], ws)
            pltpu.sync_copy(ei_hbm.at[pl.ds(e + sbase, stage)], ds_)
            pltpu.sync_copy(ei_hbm.at[pl.ds(sbase, stage)], ss)

            prep(0, sets[0])

            def rot(m, _):
                for i in range(3):
                    if i == 2:
                        wait_scatter(sets[0])
                    else:

                        @pl.when(m > 0)
                        def _():
                            wait_scatter(sets[i + 1])

                    prep(3 * m + 1 + i, sets[(i + 1) % 3])
                    finish(sets[i % 3])
                return 0

            nrot = (stage // CHUNK - 1) // 3
            lax.fori_loop(0, nrot, rot, 0)
            finish(sets[0])
            for st in sets:
                wait_scatter(st)
            return 0

        lax.fori_loop(0, eb // stage, stage_b, 0)

        plsc.subcore_barrier()

        @pl.when(c == 0)
        def _():
            pltpu.sync_copy(acc.at[pl.ds(s * rpt, rpt)],
                            out0_hbm.at[pl.ds(s * rpt, rpt)])

        @pl.when(c == 1)
        def _():
            pltpu.sync_copy(acc.at[pl.ds(s * rpt, rpt)],
                            out1_hbm.at[pl.ds(s * rpt, rpt)])

    return pl.kernel(
        body,
        out_type=[jax.ShapeDtypeStruct((npad, d), jnp.float32),
                  jax.ShapeDtypeStruct((npad, d), jnp.float32)],
        mesh=mesh,
        scratch_types=[
            pltpu.VMEM((stage,), jnp.float32),
            pltpu.VMEM((stage,), jnp.int32),
            pltpu.VMEM((stage,), jnp.int32),
            pltpu.VMEM((spt,), jnp.float32),
            pltpu.VMEM((CHUNK, d), jnp.float32),
            pltpu.VMEM((CHUNK, d), jnp.float32),
            pltpu.VMEM((CHUNK, d), jnp.float32),
            pltpu.VMEM((CHUNK,), jnp.int32),
            pltpu.VMEM((CHUNK,), jnp.int32),
            pltpu.VMEM((CHUNK,), jnp.int32),
            pltpu.VMEM((CHUNK,), jnp.int32),
            pltpu.VMEM((CHUNK,), jnp.int32),
            pltpu.VMEM((CHUNK,), jnp.int32),
            pltpu.VMEM((CHUNK,), jnp.float32),
            pltpu.VMEM((CHUNK,), jnp.float32),
            pltpu.VMEM((CHUNK,), jnp.float32),
            pltpu.VMEM((CHUNK,), jnp.int32),
            pltpu.VMEM((CHUNK,), jnp.int32),
            pltpu.VMEM((CHUNK,), jnp.int32),
            pltpu.VMEM((CHUNK,), jnp.float32),
            pltpu.VMEM((CHUNK,), jnp.float32),
            pltpu.VMEM((CHUNK,), jnp.float32),
            pltpu.SemaphoreType.DMA,
            pltpu.SemaphoreType.DMA,
            pltpu.SemaphoreType.DMA,
            pltpu.SemaphoreType.DMA,
            pltpu.SemaphoreType.DMA,
            pltpu.SemaphoreType.DMA,
            pltpu.SemaphoreType.DMA,
            pltpu.SemaphoreType.DMA,
            pltpu.SemaphoreType.DMA,
            pltpu.VMEM_SHARED((sp,), jnp.float32),
            pltpu.VMEM_SHARED((npad, d), jnp.float32),
        ],
    )


def kernel(feat, edge_index, edge_weight, W, b_fc, bias,
           coef_self, coef_posi, coef_nega):
    n, din = feat.shape
    dout = W.shape[0]
    e = edge_weight.shape[0]

    wt = W.T
    wts = jnp.concatenate(
        [wt[:din] * coef_self, wt[din:2 * din] * coef_posi,
         wt[2 * din:] * coef_nega], axis=0).reshape(3, din, dout)

    rb = 2000
    mm = pl.pallas_call(
        _mm_body,
        grid=(3, n // rb),
        in_specs=[
            pl.BlockSpec((rb, din), lambda h, i: (i, 0)),
            pl.BlockSpec((1, din, dout), lambda h, i: (h, 0, 0)),
            pl.BlockSpec((1, dout), lambda h, i: (0, 0)),
            pl.BlockSpec((1, dout), lambda h, i: (0, 0)),
        ],
        out_specs=pl.BlockSpec((rb, dout), lambda h, i: (h * (n // rb) + i, 0)),
        out_shape=jax.ShapeDtypeStruct((3 * n, dout), jnp.float32),
    )(feat, wts, b_fc.reshape(1, dout), bias.reshape(1, dout))

    sc = _make_sc_kernel(n, e, dout)
    p0, p1 = sc(mm, edge_index.reshape(2 * e), edge_weight)

    cb = 2000
    out = pl.pallas_call(
        _comb_body,
        grid=(n // cb,),
        in_specs=[
            pl.BlockSpec((cb, dout), lambda i: (i, 0)),
            pl.BlockSpec((cb, dout), lambda i: (i, 0)),
            pl.BlockSpec((cb, dout), lambda i: (i, 0)),
        ],
        out_specs=pl.BlockSpec((cb, dout), lambda i: (i, 0)),
        out_shape=jax.ShapeDtypeStruct((n, dout), jnp.float32),
    )(mm, p0, p1)
    return out


# speedup vs baseline: 52.4085x; 1.0141x over previous
import functools

import jax
import jax.numpy as jnp
from jax import lax
from jax.experimental import pallas as pl
from jax.experimental.pallas import tpu as pltpu
from jax.experimental.pallas import tpu_sc as plsc

NC = 2
NS = 16
CHUNK = 80


def _mm_body(f_ref, wt_ref, b1_ref, b2_ref, o_ref):
    h = pl.program_id(1)
    f = f_ref[...]
    o_ref[...] = jnp.dot(f, wt_ref[0], preferred_element_type=jnp.float32)

    @pl.when(h == 0)
    def _():
        o_ref[...] += b1_ref[...] + b2_ref[...]


def _comb_body(b_ref, p0_ref, p1_ref, o_ref):
    o_ref[...] = b_ref[...] + p0_ref[...] + p1_ref[...]


def _edge_vals(wv, dv, n):
    zf = jnp.zeros((16,), jnp.float32)
    zi = jnp.zeros((16,), jnp.int32)
    nvec = jnp.full((16,), n, jnp.int32)
    negv = wv < zf
    ni = jnp.where(negv, nvec, zi)
    ev = jnp.where(wv > zf, jnp.exp(wv), jnp.where(negv, jnp.exp(-wv), zf))
    return ev, dv + ni, ni


def _make_sc_kernel(n, e, d):
    ea = e // NS
    eb = e // (NC * NS)
    rpt = (-(-n // NS) + 39) // 40 * 40
    npad = NS * rpt
    sp = ((2 * n + NS * 16 - 1) // (NS * 16)) * NS * 16
    spt = sp // NS
    mesh = plsc.VectorSubcoreMesh(core_axis_name="c", subcore_axis_name="s")

    stage = 2000

    def body(g_hbm, ei_hbm, w_hbm, out0_hbm, out1_hbm,
             ws, ds_, ss, zbuf,
             rows0, rows1, rows2, sidx0, sidx1, sidx2,
             dst0, dst1, dst2, e0, e1, e2, d20, d21, d22,
             den0, den1, den2,
             gsem0, gsem1, gsem2, ssem0, ssem1, ssem2,
             dsem0, dsem1, dsem2,
             s_tab, acc):
        c = lax.axis_index("c")
        s = lax.axis_index("s")
        wid = c * NS + s

        zv = jnp.zeros((16,), jnp.float32)

        def zero_body(i, _):
            zbuf[pl.ds(i * 16, 16)] = zv
            return 0

        lax.fori_loop(0, spt // 16, zero_body, 0)
        pltpu.sync_copy(zbuf, s_tab.at[pl.ds(s * spt, spt)])

        def zrow(i, _):
            for t in range(d // 16):
                rows0[i, pl.ds(t * 16, 16)] = zv
            return 0

        lax.fori_loop(0, CHUNK, zrow, 0)
        for k in range(rpt // CHUNK):
            pltpu.sync_copy(rows0,
                            acc.at[pl.ds(s * rpt + k * CHUNK, CHUNK)])

        plsc.subcore_barrier()

        def comp_a(i, ebx, d2x):
            off = i * CHUNK
            for j in range(CHUNK // 16):
                wv = ws[pl.ds(off + j * 16, 16)]
                dv = ds_[pl.ds(off + j * 16, 16)]
                ev, d2v, _ = _edge_vals(wv, dv, n)
                ebx[pl.ds(j * 16, 16)] = ev
                d2x[pl.ds(j * 16, 16)] = d2v

        def issue_a(ebx, d2x, semx):
            pltpu.async_copy(ebx, s_tab.at[d2x], semx, add=True)

        def wait_a(ebx, d2x, semx):
            pltpu.make_async_copy(ebx, s_tab.at[d2x], semx).wait()

        def stage_a(t, _):
            sbase = s * ea + t * stage
            pltpu.sync_copy(w_hbm.at[pl.ds(sbase, stage)], ws)
            pltpu.sync_copy(ei_hbm.at[pl.ds(e + sbase, stage)], ds_)
            comp_a(0, e0, d20)
            issue_a(e0, d20, dsem0)

            def dbl(m, _):
                @pl.when(m > 0)
                def _():
                    wait_a(e1, d21, dsem1)

                comp_a(2 * m + 1, e1, d21)
                issue_a(e1, d21, dsem1)
                wait_a(e0, d20, dsem0)
                comp_a(2 * m + 2, e0, d20)
                issue_a(e0, d20, dsem0)
                return 0

            lax.fori_loop(0, (stage // CHUNK - 1) // 2, dbl, 0)
            wait_a(e0, d20, dsem0)
            wait_a(e1, d21, dsem1)
            return 0

        lax.fori_loop(0, ea // stage, stage_a, 0)

        plsc.subcore_barrier()

        tiny = jnp.full((16,), 1e-30, jnp.float32)
        sets = ((rows0, sidx0, dst0, e0, d20, den0, gsem0, ssem0, dsem0),
                (rows1, sidx1, dst1, e1, d21, den1, gsem1, ssem1, dsem1),
                (rows2, sidx2, dst2, e2, d22, den2, gsem2, ssem2, dsem2))

        def prep(k, st):
            rowsx, sidxx, dstx, ex, d2x, denx, gsemx, _, dsemx = st
            off = k * CHUNK
            for j in range(CHUNK // 16):
                wv = ws[pl.ds(off + j * 16, 16)]
                dv = ds_[pl.ds(off + j * 16, 16)]
                srcv = ss[pl.ds(off + j * 16, 16)]
                ev, d2v, ni = _edge_vals(wv, dv, n)
                ex[pl.ds(j * 16, 16)] = ev
                d2x[pl.ds(j * 16, 16)] = d2v
                sidxx[pl.ds(j * 16, 16)] = (
                    srcv + ni + jnp.full((16,), n, jnp.int32))
                dstx[pl.ds(j * 16, 16)] = dv
            pltpu.async_copy(s_tab.at[d2x], denx, dsemx)
            pltpu.async_copy(g_hbm.at[sidxx], rowsx, gsemx)

        def finish(st):
            rowsx, sidxx, dstx, ex, d2x, denx, gsemx, ssemx, dsemx = st
            pltpu.make_async_copy(s_tab.at[d2x], denx, dsemx).wait()
            pltpu.make_async_copy(g_hbm.at[sidxx], rowsx, gsemx).wait()

            def srow(j, _):
                ev16 = ex[pl.ds(j * 16, 16)]
                dn16 = denx[pl.ds(j * 16, 16)]
                sva = ev16 / jnp.maximum(dn16, tiny)
                for l in range(16):
                    sv = lax.gather(
                        sva, jnp.full((16, 1), l, jnp.int32),
                        dimension_numbers=lax.GatherDimensionNumbers(
                            offset_dims=(), collapsed_slice_dims=(0,),
                            start_index_map=(0,)),
                        slice_sizes=(1,),
                        mode=lax.GatherScatterMode.PROMISE_IN_BOUNDS)
                    r = j * 16 + l
                    for t in range(d // 16):
                        rowsx[r, pl.ds(t * 16, 16)] = (
                            rowsx[r, pl.ds(t * 16, 16)] * sv)
                return 0

            lax.fori_loop(0, CHUNK // 16, srow, 0)
            pltpu.async_copy(rowsx, acc.at[dstx], ssemx, add=True)

        def wait_scatter(st):
            rowsx, _, dstx = st[0], st[1], st[2]
            ssemx = st[7]
            pltpu.make_async_copy(rowsx, acc.at[dstx], ssemx).wait()

        def stage_b(t, _):
            sbase = wid * eb + t * stage
            pltpu.sync_copy(w_hbm.at[pl.ds(sbase, stage)], ws)
            pltpu.sync_copy(ei_hbm.at[pl.ds(e + sbase, stage)], ds_)
            pltpu.sync_copy(ei_hbm.at[pl.ds(sbase, stage)], ss)

            prep(0, sets[0])

            def rot(m, _):
                for i in range(3):
                    if i == 2:
                        wait_scatter(sets[0])
                    else:

                        @pl.when(m > 0)
                        def _():
                            wait_scatter(sets[i + 1])

                    prep(3 * m + 1 + i, sets[(i + 1) % 3])
                    finish(sets[i % 3])
                return 0

            nrot = (stage // CHUNK - 1) // 3
            lax.fori_loop(0, nrot, rot, 0)
            finish(sets[0])
            for st in sets:
                wait_scatter(st)
            return 0

        lax.fori_loop(0, eb // stage, stage_b, 0)

        plsc.subcore_barrier()

        @pl.when(c == 0)
        def _():
            pltpu.sync_copy(acc.at[pl.ds(s * rpt, rpt)],
                            out0_hbm.at[pl.ds(s * rpt, rpt)])

        @pl.when(c == 1)
        def _():
            pltpu.sync_copy(acc.at[pl.ds(s * rpt, rpt)],
                            out1_hbm.at[pl.ds(s * rpt, rpt)])

    return pl.kernel(
        body,
        out_type=[jax.ShapeDtypeStruct((npad, d), jnp.float32),
                  jax.ShapeDtypeStruct((npad, d), jnp.float32)],
        mesh=mesh,
        scratch_types=[
            pltpu.VMEM((stage,), jnp.float32),
            pltpu.VMEM((stage,), jnp.int32),
            pltpu.VMEM((stage,), jnp.int32),
            pltpu.VMEM((spt,), jnp.float32),
            pltpu.VMEM((CHUNK, d), jnp.float32),
            pltpu.VMEM((CHUNK, d), jnp.float32),
            pltpu.VMEM((CHUNK, d), jnp.float32),
            pltpu.VMEM((CHUNK,), jnp.int32),
            pltpu.VMEM((CHUNK,), jnp.int32),
            pltpu.VMEM((CHUNK,), jnp.int32),
            pltpu.VMEM((CHUNK,), jnp.int32),
            pltpu.VMEM((CHUNK,), jnp.int32),
            pltpu.VMEM((CHUNK,), jnp.int32),
            pltpu.VMEM((CHUNK,), jnp.float32),
            pltpu.VMEM((CHUNK,), jnp.float32),
            pltpu.VMEM((CHUNK,), jnp.float32),
            pltpu.VMEM((CHUNK,), jnp.int32),
            pltpu.VMEM((CHUNK,), jnp.int32),
            pltpu.VMEM((CHUNK,), jnp.int32),
            pltpu.VMEM((CHUNK,), jnp.float32),
            pltpu.VMEM((CHUNK,), jnp.float32),
            pltpu.VMEM((CHUNK,), jnp.float32),
            pltpu.SemaphoreType.DMA,
            pltpu.SemaphoreType.DMA,
            pltpu.SemaphoreType.DMA,
            pltpu.SemaphoreType.DMA,
            pltpu.SemaphoreType.DMA,
            pltpu.SemaphoreType.DMA,
            pltpu.SemaphoreType.DMA,
            pltpu.SemaphoreType.DMA,
            pltpu.SemaphoreType.DMA,
            pltpu.VMEM_SHARED((sp,), jnp.float32),
            pltpu.VMEM_SHARED((npad, d), jnp.float32),
        ],
    )


def kernel(feat, edge_index, edge_weight, W, b_fc, bias,
           coef_self, coef_posi, coef_nega):
    n, din = feat.shape
    dout = W.shape[0]
    e = edge_weight.shape[0]

    wt = W.T
    wts = jnp.concatenate(
        [wt[:din] * coef_self, wt[din:2 * din] * coef_posi,
         wt[2 * din:] * coef_nega], axis=0).reshape(3, din, dout)

    rb = 2000
    mm = pl.pallas_call(
        _mm_body,
        grid=(n // rb, 3),
        in_specs=[
            pl.BlockSpec((rb, din), lambda i, h: (i, 0)),
            pl.BlockSpec((1, din, dout), lambda i, h: (h, 0, 0)),
            pl.BlockSpec((1, dout), lambda i, h: (0, 0)),
            pl.BlockSpec((1, dout), lambda i, h: (0, 0)),
        ],
        out_specs=pl.BlockSpec((rb, dout), lambda i, h: (h * (n // rb) + i, 0)),
        out_shape=jax.ShapeDtypeStruct((3 * n, dout), jnp.float32),
    )(feat, wts, b_fc.reshape(1, dout), bias.reshape(1, dout))

    sc = _make_sc_kernel(n, e, dout)
    p0, p1 = sc(mm, edge_index.reshape(2 * e), edge_weight)

    cb = 2000
    out = pl.pallas_call(
        _comb_body,
        grid=(n // cb,),
        in_specs=[
            pl.BlockSpec((cb, dout), lambda i: (i, 0)),
            pl.BlockSpec((cb, dout), lambda i: (i, 0)),
            pl.BlockSpec((cb, dout), lambda i: (i, 0)),
        ],
        out_specs=pl.BlockSpec((cb, dout), lambda i: (i, 0)),
        out_shape=jax.ShapeDtypeStruct((n, dout), jnp.float32),
    )(mm, p0, p1)
    return out


# speedup vs baseline: 52.4288x; 1.0004x over previous
import jax
import jax.numpy as jnp
from jax import lax
from jax.experimental import pallas as pl
from jax.experimental.pallas import tpu as pltpu
from jax.experimental.pallas import tpu_sc as plsc

NC = 2
NS = 16
CHUNK = 80


def _mm_body(f_ref, wt_ref, b1_ref, b2_ref, o_ref):
    h = pl.program_id(1)
    f = f_ref[...]
    o_ref[...] = jnp.dot(f, wt_ref[0], preferred_element_type=jnp.float32)

    @pl.when(h == 0)
    def _():
        o_ref[...] += b1_ref[...] + b2_ref[...]


def _comb_body(b_ref, p0_ref, p1_ref, o_ref):
    o_ref[...] = b_ref[...] + p0_ref[...] + p1_ref[...]


def _edge_vals(wv, dv, n):
    zf = jnp.zeros((16,), jnp.float32)
    zi = jnp.zeros((16,), jnp.int32)
    nvec = jnp.full((16,), n, jnp.int32)
    negv = wv < zf
    ni = jnp.where(negv, nvec, zi)
    ev = jnp.where(wv > zf, jnp.exp(wv), jnp.where(negv, jnp.exp(-wv), zf))
    return ev, dv + ni, ni


def _make_sc_kernel(n, e, d):
    ea = e // NS
    eb = e // (NC * NS)
    rpt = (-(-n // NS) + 39) // 40 * 40
    npad = NS * rpt
    sp = ((2 * n + NS * 16 - 1) // (NS * 16)) * NS * 16
    spt = sp // NS
    mesh = plsc.VectorSubcoreMesh(core_axis_name="c", subcore_axis_name="s")

    stage = 2000

    def body(g_hbm, ei_hbm, w_hbm, out0_hbm, out1_hbm,
             ws, ds_, ss, zbuf,
             rows0, rows1, rows2, sidx0, sidx1, sidx2,
             dst0, dst1, dst2, e0, e1, e2, d20, d21, d22,
             den0, den1, den2,
             gsem0, gsem1, gsem2, ssem0, ssem1, ssem2,
             dsem0, dsem1, dsem2,
             s_tab, acc):
        c = lax.axis_index("c")
        s = lax.axis_index("s")
        wid = c * NS + s

        zv = jnp.zeros((16,), jnp.float32)

        def zero_body(i, _):
            zbuf[pl.ds(i * 16, 16)] = zv
            return 0

        lax.fori_loop(0, spt // 16, zero_body, 0)
        pltpu.sync_copy(zbuf, s_tab.at[pl.ds(s * spt, spt)])

        def zrow(i, _):
            for t in range(d // 16):
                rows0[i, pl.ds(t * 16, 16)] = zv
            return 0

        lax.fori_loop(0, CHUNK, zrow, 0)
        for k in range(rpt // CHUNK):
            pltpu.sync_copy(rows0,
                            acc.at[pl.ds(s * rpt + k * CHUNK, CHUNK)])

        plsc.subcore_barrier()

        def comp_a(i, ebx, d2x):
            off = i * CHUNK
            for j in range(CHUNK // 16):
                wv = ws[pl.ds(off + j * 16, 16)]
                dv = ds_[pl.ds(off + j * 16, 16)]
                ev, d2v, _ = _edge_vals(wv, dv, n)
                ebx[pl.ds(j * 16, 16)] = ev
                d2x[pl.ds(j * 16, 16)] = d2v

        def issue_a(ebx, d2x, semx):
            pltpu.async_copy(ebx, s_tab.at[d2x], semx, add=True)

        def wait_a(ebx, d2x, semx):
            pltpu.make_async_copy(ebx, s_tab.at[d2x], semx).wait()

        def stage_a(t, _):
            sbase = s * ea + t * stage
            pltpu.sync_copy(w_hbm.at[pl.ds(sbase, stage)], ws)
            pltpu.sync_copy(ei_hbm.at[pl.ds(e + sbase, stage)], ds_)
            comp_a(0, e0, d20)
            issue_a(e0, d20, dsem0)

            def dbl(m, _):
                @pl.when(m > 0)
                def _():
                    wait_a(e1, d21, dsem1)

                comp_a(2 * m + 1, e1, d21)
                issue_a(e1, d21, dsem1)
                wait_a(e0, d20, dsem0)
                comp_a(2 * m + 2, e0, d20)
                issue_a(e0, d20, dsem0)
                return 0

            lax.fori_loop(0, (stage // CHUNK - 1) // 2, dbl, 0)
            wait_a(e0, d20, dsem0)
            wait_a(e1, d21, dsem1)
            return 0

        lax.fori_loop(0, ea // stage, stage_a, 0)

        plsc.subcore_barrier()

        tiny = jnp.full((16,), 1e-30, jnp.float32)
        sets = ((rows0, sidx0, dst0, e0, d20, den0, gsem0, ssem0, dsem0),
                (rows1, sidx1, dst1, e1, d21, den1, gsem1, ssem1, dsem1),
                (rows2, sidx2, dst2, e2, d22, den2, gsem2, ssem2, dsem2))

        def prep(k, st):
            rowsx, sidxx, dstx, ex, d2x, denx, gsemx, _, dsemx = st
            off = k * CHUNK
            for j in range(CHUNK // 16):
                wv = ws[pl.ds(off + j * 16, 16)]
                dv = ds_[pl.ds(off + j * 16, 16)]
                srcv = ss[pl.ds(off + j * 16, 16)]
                ev, d2v, ni = _edge_vals(wv, dv, n)
                ex[pl.ds(j * 16, 16)] = ev
                d2x[pl.ds(j * 16, 16)] = d2v
                sidxx[pl.ds(j * 16, 16)] = (
                    srcv + ni + jnp.full((16,), n, jnp.int32))
                dstx[pl.ds(j * 16, 16)] = dv
            pltpu.async_copy(s_tab.at[d2x], denx, dsemx)
            pltpu.async_copy(g_hbm.at[sidxx], rowsx, gsemx)

        def finish(st):
            rowsx, sidxx, dstx, ex, d2x, denx, gsemx, ssemx, dsemx = st
            pltpu.make_async_copy(s_tab.at[d2x], denx, dsemx).wait()
            pltpu.make_async_copy(g_hbm.at[sidxx], rowsx, gsemx).wait()

            def srow(j, _):
                ev16 = ex[pl.ds(j * 16, 16)]
                dn16 = denx[pl.ds(j * 16, 16)]
                sva = ev16 / jnp.maximum(dn16, tiny)
                for l in range(16):
                    sv = lax.gather(
                        sva, jnp.full((16, 1), l, jnp.int32),
                        dimension_numbers=lax.GatherDimensionNumbers(
                            offset_dims=(), collapsed_slice_dims=(0,),
                            start_index_map=(0,)),
                        slice_sizes=(1,),
                        mode=lax.GatherScatterMode.PROMISE_IN_BOUNDS)
                    r = j * 16 + l
                    for t in range(d // 16):
                        rowsx[r, pl.ds(t * 16, 16)] = (
                            rowsx[r, pl.ds(t * 16, 16)] * sv)
                return 0

            lax.fori_loop(0, CHUNK // 16, srow, 0)
            pltpu.async_copy(rowsx, acc.at[dstx], ssemx, add=True)

        def wait_scatter(st):
            rowsx, _, dstx = st[0], st[1], st[2]
            ssemx = st[7]
            pltpu.make_async_copy(rowsx, acc.at[dstx], ssemx).wait()

        def stage_b(t, _):
            sbase = wid * eb + t * stage
            pltpu.sync_copy(w_hbm.at[pl.ds(sbase, stage)], ws)
            pltpu.sync_copy(ei_hbm.at[pl.ds(e + sbase, stage)], ds_)
            pltpu.sync_copy(ei_hbm.at[pl.ds(sbase, stage)], ss)

            prep(0, sets[0])

            def rot(m, _):
                for i in range(3):
                    if i == 2:
                        wait_scatter(sets[0])
                    else:

                        @pl.when(m > 0)
                        def _():
                            wait_scatter(sets[i + 1])

                    prep(3 * m + 1 + i, sets[(i + 1) % 3])
                    finish(sets[i % 3])
                return 0

            nrot = (stage // CHUNK - 1) // 3
            lax.fori_loop(0, nrot, rot, 0)
            finish(sets[0])
            for st in sets:
                wait_scatter(st)
            return 0

        lax.fori_loop(0, eb // stage, stage_b, 0)

        plsc.subcore_barrier()

        @pl.when(c == 0)
        def _():
            pltpu.sync_copy(acc.at[pl.ds(s * rpt, rpt)],
                            out0_hbm.at[pl.ds(s * rpt, rpt)])

        @pl.when(c == 1)
        def _():
            pltpu.sync_copy(acc.at[pl.ds(s * rpt, rpt)],
                            out1_hbm.at[pl.ds(s * rpt, rpt)])

    return pl.kernel(
        body,
        out_type=[jax.ShapeDtypeStruct((npad, d), jnp.float32),
                  jax.ShapeDtypeStruct((npad, d), jnp.float32)],
        mesh=mesh,
        scratch_types=[
            pltpu.VMEM((stage,), jnp.float32),
            pltpu.VMEM((stage,), jnp.int32),
            pltpu.VMEM((stage,), jnp.int32),
            pltpu.VMEM((spt,), jnp.float32),
            pltpu.VMEM((CHUNK, d), jnp.float32),
            pltpu.VMEM((CHUNK, d), jnp.float32),
            pltpu.VMEM((CHUNK, d), jnp.float32),
            pltpu.VMEM((CHUNK,), jnp.int32),
            pltpu.VMEM((CHUNK,), jnp.int32),
            pltpu.VMEM((CHUNK,), jnp.int32),
            pltpu.VMEM((CHUNK,), jnp.int32),
            pltpu.VMEM((CHUNK,), jnp.int32),
            pltpu.VMEM((CHUNK,), jnp.int32),
            pltpu.VMEM((CHUNK,), jnp.float32),
            pltpu.VMEM((CHUNK,), jnp.float32),
            pltpu.VMEM((CHUNK,), jnp.float32),
            pltpu.VMEM((CHUNK,), jnp.int32),
            pltpu.VMEM((CHUNK,), jnp.int32),
            pltpu.VMEM((CHUNK,), jnp.int32),
            pltpu.VMEM((CHUNK,), jnp.float32),
            pltpu.VMEM((CHUNK,), jnp.float32),
            pltpu.VMEM((CHUNK,), jnp.float32),
            pltpu.SemaphoreType.DMA,
            pltpu.SemaphoreType.DMA,
            pltpu.SemaphoreType.DMA,
            pltpu.SemaphoreType.DMA,
            pltpu.SemaphoreType.DMA,
            pltpu.SemaphoreType.DMA,
            pltpu.SemaphoreType.DMA,
            pltpu.SemaphoreType.DMA,
            pltpu.SemaphoreType.DMA,
            pltpu.VMEM_SHARED((sp,), jnp.float32),
            pltpu.VMEM_SHARED((npad, d), jnp.float32),
        ],
    )


def kernel(feat, edge_index, edge_weight, W, b_fc, bias,
           coef_self, coef_posi, coef_nega):
    n, din = feat.shape
    dout = W.shape[0]
    e = edge_weight.shape[0]

    wt = W.T
    wts = jnp.concatenate(
        [wt[:din] * coef_self, wt[din:2 * din] * coef_posi,
         wt[2 * din:] * coef_nega], axis=0).reshape(3, din, dout)

    rb = 2000
    mm = pl.pallas_call(
        _mm_body,
        grid=(n // rb, 3),
        in_specs=[
            pl.BlockSpec((rb, din), lambda i, h: (i, 0)),
            pl.BlockSpec((1, din, dout), lambda i, h: (h, 0, 0)),
            pl.BlockSpec((1, dout), lambda i, h: (0, 0)),
            pl.BlockSpec((1, dout), lambda i, h: (0, 0)),
        ],
        out_specs=pl.BlockSpec((rb, dout), lambda i, h: (h * (n // rb) + i, 0)),
        out_shape=jax.ShapeDtypeStruct((3 * n, dout), jnp.float32),
    )(feat, wts, b_fc.reshape(1, dout), bias.reshape(1, dout))

    sc = _make_sc_kernel(n, e, dout)
    p0, p1 = sc(mm, edge_index.reshape(2 * e), edge_weight)

    cb = 2000
    out = pl.pallas_call(
        _comb_body,
        grid=(n // cb,),
        in_specs=[
            pl.BlockSpec((cb, dout), lambda i: (i, 0)),
            pl.BlockSpec((cb, dout), lambda i: (i, 0)),
            pl.BlockSpec((cb, dout), lambda i: (i, 0)),
        ],
        out_specs=pl.BlockSpec((cb, dout), lambda i: (i, 0)),
        out_shape=jax.ShapeDtypeStruct((n, dout), jnp.float32),
    )(mm, p0, p1)
    return out
